# Initial kernel scaffold; baseline (speedup 1.0000x reference)
import jax
import jax.numpy as jnp
from jax.experimental import pallas as pl


def kernel(mem, data, head):
    raise NotImplementedError("write your pallas kernel here")


# speedup vs baseline: 1.9239x; 1.9239x over previous
import functools

import jax
import jax.numpy as jnp
from jax import lax
from jax.experimental import pallas as pl
from jax.experimental.pallas import tpu as pltpu
from jax.experimental.pallas import tpu_sc as plsc

_LANES = 16
_SCATTER_CHUNK = 128


@functools.lru_cache(maxsize=None)
def _make_store(buffer_size: int, n: int, d: int):
    try:
        info = plsc.get_sparse_core_info()
        nc, ns = info.num_cores, info.num_subcores
    except Exception:
        nc, ns = 2, 16
    nw = nc * ns
    rpw = n // nw
    assert rpw * nw == n and rpw % _SCATTER_CHUNK == 0
    n_chunks = rpw // _SCATTER_CHUNK

    mesh = plsc.VectorSubcoreMesh(
        core_axis_name="c", subcore_axis_name="s", num_cores=nc, num_subcores=ns
    )

    @functools.partial(
        pl.kernel,
        mesh=mesh,
        out_type=(),
        compiler_params=pltpu.CompilerParams(use_tc_tiling_on_sc=False),
        scratch_types=[
            pltpu.VMEM((_LANES,), jnp.int32),
            pltpu.VMEM((n_chunks, _SCATTER_CHUNK), jnp.int32),
            pltpu.VMEM((rpw, d), jnp.float32),
            pltpu.SemaphoreType.DMA,
        ],
    )
    def store(head_hbm, data_hbm, mem_hbm, head_v, idx_v, rows_v, sem):
        cid = lax.axis_index("c")
        sid = lax.axis_index("s")
        wid = sid * nc + cid
        base = wid * rpw

        pltpu.sync_copy(head_hbm, head_v)
        hv = head_v[...]
        lane = lax.iota(jnp.int32, _LANES)
        per_row = _SCATTER_CHUNK // _LANES
        for j in range(rpw // _LANES):
            v = (hv + (base + j * _LANES) + lane) % buffer_size
            idx_v[j // per_row, pl.ds((j % per_row) * _LANES, _LANES)] = v

        pltpu.sync_copy(data_hbm.at[pl.ds(base, rpw)], rows_v)
        for t in range(n_chunks):
            pltpu.async_copy(
                rows_v.at[pl.ds(t * _SCATTER_CHUNK, _SCATTER_CHUNK)],
                mem_hbm.at[idx_v.at[t]],
                sem,
            ).wait()

    return store


def kernel(mem, data, head):
    n, d = data.shape
    buffer_size = mem.shape[0]
    head_vec = jnp.full((_LANES,), head, dtype=jnp.int32) % buffer_size
    store = _make_store(buffer_size, n, d)
    mem_ref = jax.new_ref(mem)
    store(head_vec, data, mem_ref)
    return mem_ref[...]


# speedup vs baseline: 10.1333x; 5.2671x over previous
import functools

import jax
import jax.numpy as jnp
from jax.experimental import pallas as pl
from jax.experimental.pallas import tpu as pltpu

_LANE = 128


@functools.lru_cache(maxsize=None)
def _make_store(b: int, n: int, d: int):
    nt = pl.cdiv(b, _LANE)
    ndb = n // _LANE
    k_steps = ndb + 3

    def _tc(k, h):
        return (h[0] // _LANE + k) % nt

    def _st(k, h):
        st = (_tc(k, h) * _LANE - h[0]) % b
        return st - jnp.where(st >= b - (_LANE - 1), b, 0)

    def _blk_a(k, h):
        return jnp.clip(_st(k, h) // _LANE, 0, ndb - 1)

    def _blk_b(k, h):
        return jnp.clip(_st(k, h) // _LANE + 1, 0, ndb - 1)

    def body(h_ref, buf_ref, da_ref, db_ref, out_ref):
        k = pl.program_id(0)
        st = _st(k, h_ref)
        s = st % _LANE
        lane = jax.lax.broadcasted_iota(jnp.int32, (d, _LANE), 1)
        iv = st + lane
        valid = (iv >= 0) & (iv < n)
        ra = pltpu.roll(da_ref[...], _LANE - s, 1)
        rb = pltpu.roll(db_ref[...], _LANE - s, 1)
        shifted = jnp.where(lane < _LANE - s, ra, rb)
        out_ref[...] = jnp.where(valid, shifted, buf_ref[...])

    grid_spec = pltpu.PrefetchScalarGridSpec(
        num_scalar_prefetch=1,
        grid=(k_steps,),
        in_specs=[
            pl.BlockSpec((d, _LANE), lambda k, h: (0, _tc(k, h))),
            pl.BlockSpec((d, _LANE), lambda k, h: (0, _blk_a(k, h))),
            pl.BlockSpec((d, _LANE), lambda k, h: (0, _blk_b(k, h))),
        ],
        out_specs=pl.BlockSpec((d, _LANE), lambda k, h: (0, _tc(k, h))),
    )

    return pl.pallas_call(
        body,
        grid_spec=grid_spec,
        out_shape=jax.ShapeDtypeStruct((d, b), jnp.float32),
        input_output_aliases={1: 0},
    )


def kernel(mem, data, head):
    n, d = data.shape
    b = mem.shape[0]
    head_arr = jnp.full((1,), head, dtype=jnp.int32) % b
    out_t = _make_store(b, n, d)(head_arr, mem.T, data.T, data.T)
    return out_t.T
